# baseline (speedup 1.0000x reference)
import functools

import jax
import jax.numpy as jnp
from jax import lax
from jax.experimental import pallas as pl
from jax.experimental.pallas import tpu as pltpu

NUM_EXPERTS = 64
TOP_K = 2
HIDDEN = 768
TOKEN_BLOCK = 8192


def _router_body(hs_ref, gw_ref, logits_ref, w_ref, e_ref):
    logits_t = lax.dot_general(
        gw_ref[...], hs_ref[...],
        dimension_numbers=(((1,), (1,)), ((), ())),
        preferred_element_type=jnp.float32,
    )
    logits_ref[...] = logits_t

    t = logits_t.shape[1]
    eidx = lax.broadcasted_iota(jnp.int32, (NUM_EXPERTS, t), 0)
    neg_inf = jnp.float32(float("-inf"))

    m0 = jnp.max(logits_t, axis=0, keepdims=True)
    i0 = jnp.min(jnp.where(logits_t == m0, eidx, NUM_EXPERTS), axis=0, keepdims=True)
    masked = jnp.where(eidx == i0, neg_inf, logits_t)
    m1 = jnp.max(masked, axis=0, keepdims=True)
    i1 = jnp.min(jnp.where(masked == m1, eidx, NUM_EXPERTS), axis=0, keepdims=True)

    e = jnp.exp(m1 - m0)
    w0 = 1.0 / (1.0 + e)
    w1 = e / (1.0 + e)

    kidx = lax.broadcasted_iota(jnp.int32, (TOP_K, t), 0)
    w_ref[...] = jnp.where(kidx == 0, w0, w1)
    e_ref[...] = jnp.where(kidx == 0, i0, i1)


@jax.jit
def _router(hs2d, gw):
    n_tokens = hs2d.shape[0]
    grid = (n_tokens // TOKEN_BLOCK,)
    return pl.pallas_call(
        _router_body,
        grid=grid,
        in_specs=[
            pl.BlockSpec((TOKEN_BLOCK, HIDDEN), lambda i: (i, 0)),
            pl.BlockSpec((NUM_EXPERTS, HIDDEN), lambda i: (0, 0)),
        ],
        out_specs=[
            pl.BlockSpec((NUM_EXPERTS, TOKEN_BLOCK), lambda i: (0, i)),
            pl.BlockSpec((TOP_K, TOKEN_BLOCK), lambda i: (0, i)),
            pl.BlockSpec((TOP_K, TOKEN_BLOCK), lambda i: (0, i)),
        ],
        out_shape=[
            jax.ShapeDtypeStruct((NUM_EXPERTS, n_tokens), jnp.float32),
            jax.ShapeDtypeStruct((TOP_K, n_tokens), jnp.float32),
            jax.ShapeDtypeStruct((TOP_K, n_tokens), jnp.int32),
        ],
    )(hs2d, gw)


def kernel(hidden_states, gate_w):
    batch, seq, hidden = hidden_states.shape
    hs2d = hidden_states.reshape(batch * seq, hidden)
    logits_t, weights_t, experts_t = _router(hs2d, gate_w)
    weights = weights_t.T.reshape(batch, seq, TOP_K)
    experts = experts_t.T.reshape(batch, seq, TOP_K)
    return weights, experts, logits_t.T.reshape(batch, seq, NUM_EXPERTS)


# speedup vs baseline: 1.7199x; 1.7199x over previous
import functools

import jax
import jax.numpy as jnp
from jax import lax
from jax.experimental import pallas as pl
from jax.experimental.pallas import tpu as pltpu

NUM_EXPERTS = 64
TOP_K = 2
HIDDEN = 768
TOKEN_BLOCK = 4096


def _router_body(hs_ref, gw_ref, logits_ref, w_ref, e_ref):
    logits_t = lax.dot_general(
        gw_ref[...], hs_ref[0],
        dimension_numbers=(((1,), (1,)), ((), ())),
        preferred_element_type=jnp.float32,
    )
    logits_ref[0] = logits_t

    t = logits_t.shape[1]
    eidx = lax.broadcasted_iota(jnp.int32, (NUM_EXPERTS, t), 0)
    neg_inf = jnp.float32(float("-inf"))

    m0 = jnp.max(logits_t, axis=0, keepdims=True)
    i0 = jnp.min(jnp.where(logits_t == m0, eidx, NUM_EXPERTS), axis=0, keepdims=True)
    masked = jnp.where(eidx == i0, neg_inf, logits_t)
    m1 = jnp.max(masked, axis=0, keepdims=True)
    i1 = jnp.min(jnp.where(masked == m1, eidx, NUM_EXPERTS), axis=0, keepdims=True)

    e = jnp.exp(m1 - m0)
    w0 = 1.0 / (1.0 + e)
    w1 = e / (1.0 + e)

    kidx = lax.broadcasted_iota(jnp.int32, (TOP_K, t), 0)
    w_ref[0] = jnp.where(kidx == 0, w0, w1)
    e_ref[0] = jnp.where(kidx == 0, i0, i1)


@jax.jit
def _router(hs, gw):
    batch, seq, _ = hs.shape
    grid = (batch, seq // TOKEN_BLOCK)
    return pl.pallas_call(
        _router_body,
        grid=grid,
        in_specs=[
            pl.BlockSpec((1, TOKEN_BLOCK, HIDDEN), lambda b, j: (b, j, 0)),
            pl.BlockSpec((NUM_EXPERTS, HIDDEN), lambda b, j: (0, 0)),
        ],
        out_specs=[
            pl.BlockSpec((1, NUM_EXPERTS, TOKEN_BLOCK), lambda b, j: (b, 0, j)),
            pl.BlockSpec((1, TOP_K, TOKEN_BLOCK), lambda b, j: (b, 0, j)),
            pl.BlockSpec((1, TOP_K, TOKEN_BLOCK), lambda b, j: (b, 0, j)),
        ],
        out_shape=[
            jax.ShapeDtypeStruct((batch, NUM_EXPERTS, seq), jnp.float32),
            jax.ShapeDtypeStruct((batch, TOP_K, seq), jnp.float32),
            jax.ShapeDtypeStruct((batch, TOP_K, seq), jnp.int32),
        ],
    )(hs, gw)


def kernel(hidden_states, gate_w):
    logits_t, weights_t, experts_t = _router(hidden_states, gate_w)
    return (
        jnp.swapaxes(weights_t, 1, 2),
        jnp.swapaxes(experts_t, 1, 2),
        jnp.swapaxes(logits_t, 1, 2),
    )
